# baseline (speedup 1.0000x reference)
import functools

import jax
import jax.numpy as jnp
from jax import lax
from jax.experimental import pallas as pl
from jax.experimental.pallas import tpu as pltpu
from jax.experimental.pallas import tpu_sc as plsc

_NUM_STAGES = 10
_EMBED_DIM = 128
_BATCH = 16384
_NC = 2
_NS = 16
_L = 16
_NW = _NC * _NS
_BPW = _BATCH // _NW
_CHUNKS = _BPW // _L
_VPR = _EMBED_DIM // _L
_OH_WORDS = _BPW * _NUM_STAGES
_QUARTERS = 8
_CPQ = _CHUNKS // _QUARTERS
_RPQ = _BPW // _QUARTERS

_mesh = plsc.VectorSubcoreMesh(core_axis_name="c", subcore_axis_name="s")


@functools.partial(
    pl.kernel,
    mesh=_mesh,
    out_type=jax.ShapeDtypeStruct((_BATCH, _EMBED_DIM), jnp.float32),
    scratch_types=[
        pltpu.VMEM((_BPW,), jnp.int32),
        pltpu.VMEM((_NUM_STAGES * _EMBED_DIM,), jnp.float32),
        pltpu.VMEM((_BPW, _EMBED_DIM), jnp.float32),
        pltpu.SemaphoreType.DMA,
    ],
    compiler_params=pltpu.CompilerParams(
        needs_layout_passes=False, skip_device_barrier=True
    ),
)
def _stage_embed_kernel(stages_hbm, table_hbm, out_emb_hbm,
                        idx_v, table_v, rows_v, sem):
    wid = lax.axis_index("s") * _NC + lax.axis_index("c")
    base = wid * _BPW

    idx_cp = pltpu.async_copy(stages_hbm.at[pl.ds(base, _BPW)], idx_v, sem)
    tab_cp = pltpu.async_copy(table_hbm, table_v, sem)
    idx_cp.wait()
    tab_cp.wait()

    def chunk_body(c, carry):
        s_chunk = idx_v[pl.ds(c * _L, _L)]
        srcs = [s_chunk[k] * _EMBED_DIM for k in range(_L)]
        prev = None
        for k in range(_L + 1):
            cur = []
            for v in range(_VPR):
                if k < _L:
                    cur.append(table_v[pl.ds(srcs[k] + v * _L, _L)])
                if prev is not None:
                    rows_v[c * _L + k - 1, pl.ds(v * _L, _L)] = prev[v]
            prev = cur

        @pl.when(c % _CPQ == _CPQ - 1)
        def _():
            q = c // _CPQ
            pltpu.make_async_copy(
                rows_v.at[pl.ds(q * _RPQ, _RPQ)],
                out_emb_hbm.at[pl.ds(base + q * _RPQ, _RPQ)],
                sem,
            ).start()

        return carry

    lax.fori_loop(0, _CHUNKS, chunk_body, 0)
    for q in range(_QUARTERS):
        pltpu.make_async_copy(
            rows_v.at[pl.ds(q * _RPQ, _RPQ)],
            out_emb_hbm.at[pl.ds(base + q * _RPQ, _RPQ)],
            sem,
        ).wait()


def _onehot_body(stages_ref, out_ref):
    s = stages_ref[...]
    stage_ids = lax.broadcasted_iota(jnp.int32, (_NUM_STAGES, _BATCH), 0)
    out_ref[...] = jnp.where(stage_ids == s, 1.0, 0.0).astype(jnp.float32)


_onehot_tc = pl.pallas_call(
    _onehot_body,
    out_shape=jax.ShapeDtypeStruct((_NUM_STAGES, _BATCH), jnp.float32),
)


def kernel(stages, table):
    stages_i32 = stages.reshape(-1).astype(jnp.int32)
    emb = _stage_embed_kernel(stages_i32, table.reshape(-1))
    oh_t = _onehot_tc(stages_i32.reshape(1, _BATCH))
    return emb, oh_t.T


# speedup vs baseline: 1.0320x; 1.0320x over previous
import functools

import jax
import jax.numpy as jnp
from jax import lax
from jax.experimental import pallas as pl
from jax.experimental.pallas import tpu as pltpu
from jax.experimental.pallas import tpu_sc as plsc

_NUM_STAGES = 10
_EMBED_DIM = 128
_BATCH = 16384
_NC = 2
_NS = 16
_L = 16
_NW = _NC * _NS
_BPW = _BATCH // _NW
_CHUNKS = _BPW // _L
_VPR = _EMBED_DIM // _L
_OH_WORDS = _BPW * _NUM_STAGES
_QUARTERS = 8
_CPQ = _CHUNKS // _QUARTERS
_RPQ = _BPW // _QUARTERS

_NREP = 8

_mesh = plsc.VectorSubcoreMesh(core_axis_name="c", subcore_axis_name="s")


@functools.partial(
    pl.kernel,
    mesh=_mesh,
    out_type=jax.ShapeDtypeStruct((_BATCH, _EMBED_DIM), jnp.float32),
    scratch_types=[
        pltpu.VMEM((_BPW,), jnp.int32),
        pltpu.VMEM((_NUM_STAGES * _EMBED_DIM,), jnp.float32),
        pltpu.VMEM((_BPW, _EMBED_DIM), jnp.float32),
        pltpu.SemaphoreType.DMA,
    ],
    compiler_params=pltpu.CompilerParams(
        needs_layout_passes=False, skip_device_barrier=True
    ),
)
def _stage_embed_kernel(stages_hbm, table_hbm, out_emb_hbm,
                        idx_v, table_v, rows_v, sem):
    wid = lax.axis_index("s") * _NC + lax.axis_index("c")
    base = wid * _BPW

    idx_cp = pltpu.async_copy(stages_hbm.at[pl.ds(base, _BPW)], idx_v, sem)
    tab_cp = pltpu.async_copy(
        table_hbm.at[pl.ds((wid % _NREP) * (_NUM_STAGES * _EMBED_DIM),
                           _NUM_STAGES * _EMBED_DIM)],
        table_v, sem,
    )
    idx_cp.wait()
    tab_cp.wait()

    def chunk_body(c, carry):
        s_chunk = idx_v[pl.ds(c * _L, _L)]
        srcs = [s_chunk[k] * _EMBED_DIM for k in range(_L)]
        prev = None
        for k in range(_L + 1):
            cur = []
            for v in range(_VPR):
                if k < _L:
                    cur.append(table_v[pl.ds(srcs[k] + v * _L, _L)])
                if prev is not None:
                    rows_v[c * _L + k - 1, pl.ds(v * _L, _L)] = prev[v]
            prev = cur

        @pl.when(c % _CPQ == _CPQ - 1)
        def _():
            q = c // _CPQ
            pltpu.make_async_copy(
                rows_v.at[pl.ds(q * _RPQ, _RPQ)],
                out_emb_hbm.at[pl.ds(base + q * _RPQ, _RPQ)],
                sem,
            ).start()

        return carry

    lax.fori_loop(0, _CHUNKS, chunk_body, 0)
    for q in range(_QUARTERS):
        pltpu.make_async_copy(
            rows_v.at[pl.ds(q * _RPQ, _RPQ)],
            out_emb_hbm.at[pl.ds(base + q * _RPQ, _RPQ)],
            sem,
        ).wait()


def _onehot_body(stages_ref, out_ref):
    s = stages_ref[...]
    stage_ids = lax.broadcasted_iota(jnp.int32, (_NUM_STAGES, _BATCH), 0)
    out_ref[...] = jnp.where(stage_ids == s, 1.0, 0.0).astype(jnp.float32)


_onehot_tc = pl.pallas_call(
    _onehot_body,
    out_shape=jax.ShapeDtypeStruct((_NUM_STAGES, _BATCH), jnp.float32),
)


def kernel(stages, table):
    stages_i32 = stages.reshape(-1).astype(jnp.int32)
    table_rep = jnp.tile(table.reshape(-1), _NREP)
    emb = _stage_embed_kernel(stages_i32, table_rep)
    oh_t = _onehot_tc(stages_i32.reshape(1, _BATCH))
    return emb, oh_t.T


# speedup vs baseline: 1.0323x; 1.0002x over previous
import functools

import jax
import jax.numpy as jnp
from jax import lax
from jax.experimental import pallas as pl
from jax.experimental.pallas import tpu as pltpu
from jax.experimental.pallas import tpu_sc as plsc

_NUM_STAGES = 10
_EMBED_DIM = 128
_BATCH = 16384
_NC = 2
_NS = 16
_L = 16
_NW = _NC * _NS
_BPW = _BATCH // _NW
_CHUNKS = _BPW // _L
_VPR = _EMBED_DIM // _L
_OH_WORDS = _BPW * _NUM_STAGES
_QUARTERS = 16
_CPQ = _CHUNKS // _QUARTERS
_RPQ = _BPW // _QUARTERS

_NREP = 8

_mesh = plsc.VectorSubcoreMesh(core_axis_name="c", subcore_axis_name="s")


@functools.partial(
    pl.kernel,
    mesh=_mesh,
    out_type=jax.ShapeDtypeStruct((_BATCH, _EMBED_DIM), jnp.float32),
    scratch_types=[
        pltpu.VMEM((_BPW,), jnp.int32),
        pltpu.VMEM((_NUM_STAGES * _EMBED_DIM,), jnp.float32),
        pltpu.VMEM((_BPW, _EMBED_DIM), jnp.float32),
        pltpu.SemaphoreType.DMA,
    ],
    compiler_params=pltpu.CompilerParams(
        needs_layout_passes=False, skip_device_barrier=True
    ),
)
def _stage_embed_kernel(stages_hbm, table_hbm, out_emb_hbm,
                        idx_v, table_v, rows_v, sem):
    wid = lax.axis_index("s") * _NC + lax.axis_index("c")
    base = wid * _BPW

    idx_cp = pltpu.async_copy(stages_hbm.at[pl.ds(base, _BPW)], idx_v, sem)
    tab_cp = pltpu.async_copy(
        table_hbm.at[pl.ds((wid % _NREP) * (_NUM_STAGES * _EMBED_DIM),
                           _NUM_STAGES * _EMBED_DIM)],
        table_v, sem,
    )
    idx_cp.wait()
    tab_cp.wait()

    def chunk_body(c, carry):
        s_chunk = idx_v[pl.ds(c * _L, _L)]
        srcs = [s_chunk[k] * _EMBED_DIM for k in range(_L)]
        prev = None
        for k in range(_L + 1):
            cur = []
            for v in range(_VPR):
                if k < _L:
                    cur.append(table_v[pl.ds(srcs[k] + v * _L, _L)])
                if prev is not None:
                    rows_v[c * _L + k - 1, pl.ds(v * _L, _L)] = prev[v]
            prev = cur

        @pl.when(c % _CPQ == _CPQ - 1)
        def _():
            q = c // _CPQ
            pltpu.make_async_copy(
                rows_v.at[pl.ds(q * _RPQ, _RPQ)],
                out_emb_hbm.at[pl.ds(base + q * _RPQ, _RPQ)],
                sem,
            ).start()

        return carry

    lax.fori_loop(0, _CHUNKS, chunk_body, 0)
    for q in range(_QUARTERS):
        pltpu.make_async_copy(
            rows_v.at[pl.ds(q * _RPQ, _RPQ)],
            out_emb_hbm.at[pl.ds(base + q * _RPQ, _RPQ)],
            sem,
        ).wait()


def _onehot_body(stages_ref, out_ref):
    s = stages_ref[...]
    stage_ids = lax.broadcasted_iota(jnp.int32, (_NUM_STAGES, _BATCH), 0)
    out_ref[...] = jnp.where(stage_ids == s, 1.0, 0.0).astype(jnp.float32)


_onehot_tc = pl.pallas_call(
    _onehot_body,
    out_shape=jax.ShapeDtypeStruct((_NUM_STAGES, _BATCH), jnp.float32),
)


def kernel(stages, table):
    stages_i32 = stages.reshape(-1).astype(jnp.int32)
    table_rep = jnp.tile(table.reshape(-1), _NREP)
    emb = _stage_embed_kernel(stages_i32, table_rep)
    oh_t = _onehot_tc(stages_i32.reshape(1, _BATCH))
    return emb, oh_t.T
